# baseline (speedup 1.0000x reference)
import functools

import jax
import jax.numpy as jnp
from jax import lax
from jax.experimental import pallas as pl
from jax.experimental.pallas import tpu as pltpu
from jax.experimental.pallas import tpu_sc as plsc

E = 8
TOPK = 2
D = 1024
DFF = 512
SHARED_FF = 1024
SCALE = 2.0
EPS = 1e-20

N = 2048
NP = N * TOPK
BM = 256
NBLK = (NP + E * BM) // BM
M_PAD = NBLK * BM
CHUNK = 128
NCH = N // CHUNK
BT = 256
NW = 32
PW = NP // NW
TW = N // NW
TC = 16


def _route_body(x_ref, gw_ref, w_out, slot_out, meta_out):
    x = x_ref[...]
    gw = gw_ref[...]
    logits = lax.dot_general(gw, x, (((1,), (1,)), ((), ())),
                             preferred_element_type=jnp.float32)
    s = jax.nn.sigmoid(logits)
    sub = lax.broadcasted_iota(jnp.int32, (E, N), 0)
    m1 = jnp.max(s, axis=0, keepdims=True)
    a1 = jnp.min(jnp.where(s == m1, sub, E), axis=0, keepdims=True)
    s2 = jnp.where(sub == a1, -1.0, s)
    m2 = jnp.max(s2, axis=0, keepdims=True)
    a2 = jnp.min(jnp.where(s2 == m2, sub, E), axis=0, keepdims=True)
    denom = m1 + m2 + EPS
    w_out[0:1, :] = m1 / denom * SCALE
    w_out[1:2, :] = m2 / denom * SCALE

    sub16 = lax.broadcasted_iota(jnp.int32, (2 * E, N), 0)
    asel = jnp.where(sub16 < E, a1, a2)
    oh2 = jnp.where((sub16 % E) == asel, 1.0, 0.0)

    lane = lax.broadcasted_iota(jnp.int32, (2 * E, N), 1)
    p = oh2
    sh = 1
    while sh < N:
        p = p + jnp.where(lane >= sh, pltpu.roll(p, sh, 1), 0.0)
        sh *= 2
    excl = p - oh2
    tot = p[:, N - 1:N]
    cnt0 = tot[0:E]
    cnt = cnt0 + tot[E:2 * E]
    carry = jnp.concatenate([jnp.zeros((E, 1), jnp.float32), cnt0], axis=0)
    rank = excl + carry

    nb = jnp.floor((cnt + (BM - 1)) * (1.0 / BM))
    li = lax.broadcasted_iota(jnp.int32, (E, E), 0)
    lj = lax.broadcasted_iota(jnp.int32, (E, E), 1)
    ltri = jnp.where(lj < li, 1.0, 0.0)
    blk_off = lax.dot_general(ltri, nb, (((1,), (0,)), ((), ())),
                              preferred_element_type=jnp.float32)
    off_pad = blk_off * BM
    val = rank + jnp.concatenate([off_pad, off_pad], axis=0)
    slot0 = jnp.sum(oh2[0:E] * val[0:E], axis=0, keepdims=True)
    slot1 = jnp.sum(oh2[E:2 * E] * val[E:2 * E], axis=0, keepdims=True)
    slot_out[0:1, :] = slot0.astype(jnp.int32)
    slot_out[1:2, :] = slot1.astype(jnp.int32)

    bi = lax.broadcasted_iota(jnp.int32, (1, 128), 1).astype(jnp.float32)
    be = jnp.sum(jnp.where(blk_off <= bi, 1.0, 0.0), axis=0, keepdims=True) - 1.0
    total = jnp.sum(nb)
    meta_out[0:1, :] = be.astype(jnp.int32)
    meta_out[1:2, :] = jnp.where(bi < total, 1, 0).astype(jnp.int32)


def _route(x, gate_w):
    return pl.pallas_call(
        _route_body,
        grid=(1,),
        in_specs=[
            pl.BlockSpec((N, D), lambda i: (0, 0)),
            pl.BlockSpec((E, D), lambda i: (0, 0)),
        ],
        out_specs=[
            pl.BlockSpec((8, N), lambda i: (0, 0)),
            pl.BlockSpec((8, N), lambda i: (0, 0)),
            pl.BlockSpec((8, 128), lambda i: (0, 0)),
        ],
        out_shape=[
            jax.ShapeDtypeStruct((8, N), jnp.float32),
            jax.ShapeDtypeStruct((8, N), jnp.int32),
            jax.ShapeDtypeStruct((8, 128), jnp.int32),
        ],
    )(x, gate_w)


@functools.lru_cache(maxsize=None)
def _sc_dispatch():
    mesh = plsc.VectorSubcoreMesh(core_axis_name="c", subcore_axis_name="s")

    HC = TW // 2

    @functools.partial(
        pl.kernel,
        out_type=jax.ShapeDtypeStruct((M_PAD, D), jnp.float32),
        mesh=mesh,
        scratch_types=[
            pltpu.VMEM((2, 2, HC), jnp.int32),
            pltpu.VMEM((2, HC, D), jnp.float32),
            pltpu.SemaphoreType.DMA,
            pltpu.SemaphoreType.DMA,
        ],
    )
    def _dispatch(x_hbm, slot_hbm, xs_hbm, idx_v, rows_v, semi, semo):
        wid = lax.axis_index("s") * 2 + lax.axis_index("c")
        t0 = wid * TW
        pltpu.sync_copy(slot_hbm.at[wid], idx_v)
        cpi = pltpu.async_copy(x_hbm.at[pl.ds(t0, HC)], rows_v.at[0], semi)
        scat = []
        for c in range(2):
            cpi.wait()
            scat.append(pltpu.async_copy(rows_v.at[c],
                                         xs_hbm.at[idx_v.at[c, 0]], semo))
            scat.append(pltpu.async_copy(rows_v.at[c],
                                         xs_hbm.at[idx_v.at[c, 1]], semo))
            if c == 0:
                cpi = pltpu.async_copy(x_hbm.at[pl.ds(t0 + HC, HC)],
                                       rows_v.at[1], semi)
        for cp in scat:
            cp.wait()

    return _dispatch


def _mm(a, b):
    return lax.dot_general(a, b, (((1,), (0,)), ((), ())),
                           preferred_element_type=jnp.float32)


def _group_body(be_ref, act_ref, xs_ref, wg_ref, wu_ref, wd_ref, out_ref):
    i = pl.program_id(0)

    @pl.when(act_ref[i] == 1)
    def _():
        x = xs_ref[...]
        h = jax.nn.silu(_mm(x, wg_ref[0])) * _mm(x, wu_ref[0])
        out_ref[...] = _mm(h, wd_ref[0])


def _grouped_mlp(be, act, xs, w_gate, w_up, w_down):
    grid_spec = pltpu.PrefetchScalarGridSpec(
        num_scalar_prefetch=2,
        grid=(NBLK,),
        in_specs=[
            pl.BlockSpec((BM, D), lambda i, be, act: (i, 0)),
            pl.BlockSpec((1, D, DFF), lambda i, be, act: (be[i], 0, 0)),
            pl.BlockSpec((1, D, DFF), lambda i, be, act: (be[i], 0, 0)),
            pl.BlockSpec((1, DFF, D), lambda i, be, act: (be[i], 0, 0)),
        ],
        out_specs=pl.BlockSpec((BM, D), lambda i, be, act: (i, 0)),
    )
    return pl.pallas_call(
        _group_body,
        grid_spec=grid_spec,
        out_shape=jax.ShapeDtypeStruct((M_PAD, D), jnp.float32),
    )(be, act, xs, w_gate, w_up, w_down)


def _shared_body(x_ref, swg_ref, swu_ref, swd_ref, out_ref):
    x = x_ref[...]
    h = jax.nn.silu(_mm(x, swg_ref[...])) * _mm(x, swu_ref[...])
    out_ref[...] = _mm(h, swd_ref[...])


def _add_body(a_ref, b_ref, o_ref):
    o_ref[...] = a_ref[...] + b_ref[...]


def _final_add(a, b):
    return pl.pallas_call(
        _add_body,
        grid=(N // BT,),
        in_specs=[
            pl.BlockSpec((BT, D), lambda i: (i, 0)),
            pl.BlockSpec((BT, D), lambda i: (i, 0)),
        ],
        out_specs=pl.BlockSpec((BT, D), lambda i: (i, 0)),
        out_shape=jax.ShapeDtypeStruct((N, D), jnp.float32),
    )(a, b)


def _shared_mlp(x, sw_gate, sw_up, sw_down):
    return pl.pallas_call(
        _shared_body,
        grid=(N // BT,),
        in_specs=[
            pl.BlockSpec((BT, D), lambda b: (b, 0)),
            pl.BlockSpec((D, SHARED_FF), lambda b: (0, 0)),
            pl.BlockSpec((D, SHARED_FF), lambda b: (0, 0)),
            pl.BlockSpec((SHARED_FF, D), lambda b: (0, 0)),
        ],
        out_specs=pl.BlockSpec((BT, D), lambda b: (b, 0)),
        out_shape=jax.ShapeDtypeStruct((N, D), jnp.float32),
    )(x, sw_gate, sw_up, sw_down)


def _splat_lane(vec, i):
    iv = jnp.full((16, 1), i, jnp.int32)
    dn = lax.GatherDimensionNumbers(offset_dims=(), collapsed_slice_dims=(0,),
                                    start_index_map=(0,))
    return lax.gather(vec, iv, dn, (1,),
                      mode=lax.GatherScatterMode.PROMISE_IN_BOUNDS)


@functools.lru_cache(maxsize=None)
def _sc_combine():
    mesh = plsc.VectorSubcoreMesh(core_axis_name="c", subcore_axis_name="s")

    @functools.partial(
        pl.kernel,
        out_type=jax.ShapeDtypeStruct((N, D), jnp.float32),
        mesh=mesh,
        scratch_types=[
            pltpu.VMEM((TW,), jnp.int32),
            pltpu.VMEM((TW,), jnp.int32),
            pltpu.VMEM((TW,), jnp.float32),
            pltpu.VMEM((TW,), jnp.float32),
            pltpu.VMEM((3, TC, D), jnp.float32),
            pltpu.VMEM((3, TC, D), jnp.float32),
            pltpu.VMEM((TC, D), jnp.float32),
            pltpu.SemaphoreType.DMA,
            pltpu.SemaphoreType.DMA,
            pltpu.SemaphoreType.DMA,
        ],
    )
    def _combine(outs_hbm, slot_hbm, w_hbm, y_hbm,
                 idx0_v, idx1_v, w0_v, w1_v, r0_v, r1_v, y_v, sem0, sem1, sem2):
        wid = lax.axis_index("s") * 2 + lax.axis_index("c")
        t0 = wid * TW
        pltpu.sync_copy(slot_hbm.at[pl.ds(t0, TW)], idx0_v)
        pltpu.sync_copy(slot_hbm.at[pl.ds(N + t0, TW)], idx1_v)
        pltpu.sync_copy(w_hbm.at[pl.ds(t0, TW)], w0_v)
        pltpu.sync_copy(w_hbm.at[pl.ds(N + t0, TW)], w1_v)

        nchunk = TW // TC
        sems = (sem0, sem1, sem2)

        def start(c):
            b = c % 3
            i0 = idx0_v[pl.ds(c * TC, TC)]
            i1 = idx1_v[pl.ds(c * TC, TC)]
            return (
                pltpu.async_copy(outs_hbm.at[i0], r0_v.at[b], sems[b]),
                pltpu.async_copy(outs_hbm.at[i1], r1_v.at[b], sems[b]),
            )

        pend = [start(0), start(1)]
        for c in range(nchunk):
            b = c % 3
            if c + 2 < nchunk:
                pend.append(start(c + 2))
            cps = pend.pop(0)
            for cp in cps:
                cp.wait()
            w0c = w0_v[pl.ds(c * TC, TC)]
            w1c = w1_v[pl.ds(c * TC, TC)]

            def tok(i, carry):
                w0s = _splat_lane(w0c, i)
                w1s = _splat_lane(w1c, i)
                for j in range(D // 16):
                    sl = pl.ds(j * 16, 16)
                    y_v[i, sl] = w0s * r0_v[b, i, sl] + w1s * r1_v[b, i, sl]
                return carry

            lax.fori_loop(0, TC, tok, 0)
            pltpu.sync_copy(y_v, y_hbm.at[pl.ds(t0 + c * TC, TC)])

    return _combine


def kernel(hidden_states, gate_w, w_gate, w_up, w_down, sw_gate, sw_up, sw_down):
    b, s, d = hidden_states.shape
    x = hidden_states.reshape(b * s, d)

    w8, slot8, meta8 = _route(x, gate_w)
    slot_flat = slot8[:TOPK].reshape(NP)
    w_flat = w8[:TOPK].reshape(NP)
    be = meta8[0, :NBLK]
    act = meta8[1, :NBLK]

    slotc = jnp.transpose(slot8[:TOPK].reshape(TOPK, NW, 2, TW // 2),
                          (1, 2, 0, 3))
    xs = _sc_dispatch()(x, slotc)
    outs = _grouped_mlp(be, act, xs, w_gate, w_up, w_down)
    sh = _shared_mlp(x, sw_gate, sw_up, sw_down)
    yr = _sc_combine()(outs, slot_flat, w_flat)
    y = _final_add(sh, yr)
    return y.reshape(b, s, d)


# speedup vs baseline: 1.0021x; 1.0021x over previous
import functools

import jax
import jax.numpy as jnp
from jax import lax
from jax.experimental import pallas as pl
from jax.experimental.pallas import tpu as pltpu
from jax.experimental.pallas import tpu_sc as plsc

E = 8
TOPK = 2
D = 1024
DFF = 512
SHARED_FF = 1024
SCALE = 2.0
EPS = 1e-20

N = 2048
NP = N * TOPK
BM = 256
NBLK = (NP + E * BM) // BM
M_PAD = NBLK * BM
CHUNK = 128
NCH = N // CHUNK
BT = 256
NW = 32
PW = NP // NW
TW = N // NW
TC = 16


def _route_body(x_ref, gw_ref, w_out, slot_out, meta_out):
    x = x_ref[...]
    gw = gw_ref[...]
    logits = lax.dot_general(gw, x, (((1,), (1,)), ((), ())),
                             preferred_element_type=jnp.float32)
    s = jax.nn.sigmoid(logits)
    sub = lax.broadcasted_iota(jnp.int32, (E, N), 0)
    m1 = jnp.max(s, axis=0, keepdims=True)
    a1 = jnp.min(jnp.where(s == m1, sub, E), axis=0, keepdims=True)
    s2 = jnp.where(sub == a1, -1.0, s)
    m2 = jnp.max(s2, axis=0, keepdims=True)
    a2 = jnp.min(jnp.where(s2 == m2, sub, E), axis=0, keepdims=True)
    denom = m1 + m2 + EPS
    w_out[0:1, :] = m1 / denom * SCALE
    w_out[1:2, :] = m2 / denom * SCALE

    sub16 = lax.broadcasted_iota(jnp.int32, (2 * E, N), 0)
    asel = jnp.where(sub16 < E, a1, a2)
    oh2 = jnp.where((sub16 % E) == asel, 1.0, 0.0)

    lane = lax.broadcasted_iota(jnp.int32, (2 * E, N), 1)
    p = oh2
    sh = 1
    while sh < N:
        p = p + jnp.where(lane >= sh, pltpu.roll(p, sh, 1), 0.0)
        sh *= 2
    excl = p - oh2
    tot = p[:, N - 1:N]
    cnt0 = tot[0:E]
    cnt = cnt0 + tot[E:2 * E]
    carry = jnp.concatenate([jnp.zeros((E, 1), jnp.float32), cnt0], axis=0)
    rank = excl + carry

    nb = jnp.floor((cnt + (BM - 1)) * (1.0 / BM))
    li = lax.broadcasted_iota(jnp.int32, (E, E), 0)
    lj = lax.broadcasted_iota(jnp.int32, (E, E), 1)
    ltri = jnp.where(lj < li, 1.0, 0.0)
    blk_off = lax.dot_general(ltri, nb, (((1,), (0,)), ((), ())),
                              preferred_element_type=jnp.float32)
    off_pad = blk_off * BM
    val = rank + jnp.concatenate([off_pad, off_pad], axis=0)
    slot0 = jnp.sum(oh2[0:E] * val[0:E], axis=0, keepdims=True)
    slot1 = jnp.sum(oh2[E:2 * E] * val[E:2 * E], axis=0, keepdims=True)
    slot_out[0:1, :] = slot0.astype(jnp.int32)
    slot_out[1:2, :] = slot1.astype(jnp.int32)

    bi = lax.broadcasted_iota(jnp.int32, (1, 128), 1).astype(jnp.float32)
    be = jnp.sum(jnp.where(blk_off <= bi, 1.0, 0.0), axis=0, keepdims=True) - 1.0
    total = jnp.sum(nb)
    meta_out[0:1, :] = be.astype(jnp.int32)
    meta_out[1:2, :] = jnp.where(bi < total, 1, 0).astype(jnp.int32)


def _route(x, gate_w):
    return pl.pallas_call(
        _route_body,
        grid=(1,),
        in_specs=[
            pl.BlockSpec((N, D), lambda i: (0, 0)),
            pl.BlockSpec((E, D), lambda i: (0, 0)),
        ],
        out_specs=[
            pl.BlockSpec((8, N), lambda i: (0, 0)),
            pl.BlockSpec((8, N), lambda i: (0, 0)),
            pl.BlockSpec((8, 128), lambda i: (0, 0)),
        ],
        out_shape=[
            jax.ShapeDtypeStruct((8, N), jnp.float32),
            jax.ShapeDtypeStruct((8, N), jnp.int32),
            jax.ShapeDtypeStruct((8, 128), jnp.int32),
        ],
    )(x, gate_w)


@functools.lru_cache(maxsize=None)
def _sc_dispatch():
    mesh = plsc.VectorSubcoreMesh(core_axis_name="c", subcore_axis_name="s")

    HC = TW // 2

    @functools.partial(
        pl.kernel,
        out_type=jax.ShapeDtypeStruct((M_PAD, D), jnp.float32),
        mesh=mesh,
        scratch_types=[
            pltpu.VMEM((2, 2, HC), jnp.int32),
            pltpu.VMEM((2, HC, D), jnp.float32),
            pltpu.SemaphoreType.DMA,
            pltpu.SemaphoreType.DMA,
        ],
    )
    def _dispatch(x_hbm, slot_hbm, xs_hbm, idx_v, rows_v, semi, semo):
        wid = lax.axis_index("s") * 2 + lax.axis_index("c")
        t0 = wid * TW
        pltpu.sync_copy(slot_hbm.at[wid], idx_v)
        cpi = pltpu.async_copy(x_hbm.at[pl.ds(t0, HC)], rows_v.at[0], semi)
        scat = []
        for c in range(2):
            cpi.wait()
            scat.append(pltpu.async_copy(rows_v.at[c],
                                         xs_hbm.at[idx_v.at[c, 0]], semo))
            scat.append(pltpu.async_copy(rows_v.at[c],
                                         xs_hbm.at[idx_v.at[c, 1]], semo))
            if c == 0:
                cpi = pltpu.async_copy(x_hbm.at[pl.ds(t0 + HC, HC)],
                                       rows_v.at[1], semi)
        for cp in scat:
            cp.wait()

    return _dispatch


def _mm(a, b):
    return lax.dot_general(a, b, (((1,), (0,)), ((), ())),
                           preferred_element_type=jnp.float32)


def _group_body(be_ref, act_ref, xs_ref, wg_ref, wu_ref, wd_ref, out_ref):
    i = pl.program_id(0)

    @pl.when(act_ref[i] == 1)
    def _():
        x = xs_ref[...]
        h = jax.nn.silu(_mm(x, wg_ref[0])) * _mm(x, wu_ref[0])
        out_ref[...] = _mm(h, wd_ref[0])


def _grouped_mlp(be, act, xs, w_gate, w_up, w_down):
    grid_spec = pltpu.PrefetchScalarGridSpec(
        num_scalar_prefetch=2,
        grid=(NBLK,),
        in_specs=[
            pl.BlockSpec((BM, D), lambda i, be, act: (i, 0)),
            pl.BlockSpec((1, D, DFF), lambda i, be, act: (be[i], 0, 0)),
            pl.BlockSpec((1, D, DFF), lambda i, be, act: (be[i], 0, 0)),
            pl.BlockSpec((1, DFF, D), lambda i, be, act: (be[i], 0, 0)),
        ],
        out_specs=pl.BlockSpec((BM, D), lambda i, be, act: (i, 0)),
    )
    return pl.pallas_call(
        _group_body,
        grid_spec=grid_spec,
        out_shape=jax.ShapeDtypeStruct((M_PAD, D), jnp.float32),
    )(be, act, xs, w_gate, w_up, w_down)


def _shared_body(x_ref, swg_ref, swu_ref, swd_ref, out_ref):
    x = x_ref[...]
    h = jax.nn.silu(_mm(x, swg_ref[...])) * _mm(x, swu_ref[...])
    out_ref[...] = _mm(h, swd_ref[...])


def _add_body(a_ref, b_ref, o_ref):
    o_ref[...] = a_ref[...] + b_ref[...]


def _final_add(a, b):
    return pl.pallas_call(
        _add_body,
        grid=(N // BT,),
        in_specs=[
            pl.BlockSpec((BT, D), lambda i: (i, 0)),
            pl.BlockSpec((BT, D), lambda i: (i, 0)),
        ],
        out_specs=pl.BlockSpec((BT, D), lambda i: (i, 0)),
        out_shape=jax.ShapeDtypeStruct((N, D), jnp.float32),
    )(a, b)


def _shared_mlp(x, sw_gate, sw_up, sw_down):
    return pl.pallas_call(
        _shared_body,
        grid=(N // BT,),
        in_specs=[
            pl.BlockSpec((BT, D), lambda b: (b, 0)),
            pl.BlockSpec((D, SHARED_FF), lambda b: (0, 0)),
            pl.BlockSpec((D, SHARED_FF), lambda b: (0, 0)),
            pl.BlockSpec((SHARED_FF, D), lambda b: (0, 0)),
        ],
        out_specs=pl.BlockSpec((BT, D), lambda b: (b, 0)),
        out_shape=jax.ShapeDtypeStruct((N, D), jnp.float32),
    )(x, sw_gate, sw_up, sw_down)


def _splat_lane(vec, i):
    iv = jnp.full((16, 1), i, jnp.int32)
    dn = lax.GatherDimensionNumbers(offset_dims=(), collapsed_slice_dims=(0,),
                                    start_index_map=(0,))
    return lax.gather(vec, iv, dn, (1,),
                      mode=lax.GatherScatterMode.PROMISE_IN_BOUNDS)


@functools.lru_cache(maxsize=None)
def _sc_combine():
    mesh = plsc.VectorSubcoreMesh(core_axis_name="c", subcore_axis_name="s")

    @functools.partial(
        pl.kernel,
        out_type=jax.ShapeDtypeStruct((N, D), jnp.float32),
        mesh=mesh,
        scratch_types=[
            pltpu.VMEM((TW,), jnp.int32),
            pltpu.VMEM((TW,), jnp.int32),
            pltpu.VMEM((TW,), jnp.float32),
            pltpu.VMEM((TW,), jnp.float32),
            pltpu.VMEM((2, TC, D), jnp.float32),
            pltpu.VMEM((2, TC, D), jnp.float32),
            pltpu.VMEM((TC, D), jnp.float32),
            pltpu.SemaphoreType.DMA,
            pltpu.SemaphoreType.DMA,
        ],
    )
    def _combine(outs_hbm, slot_hbm, w_hbm, y_hbm,
                 idx0_v, idx1_v, w0_v, w1_v, r0_v, r1_v, y_v, sem0, sem1):
        wid = lax.axis_index("s") * 2 + lax.axis_index("c")
        t0 = wid * TW
        pltpu.sync_copy(slot_hbm.at[pl.ds(t0, TW)], idx0_v)
        pltpu.sync_copy(slot_hbm.at[pl.ds(N + t0, TW)], idx1_v)
        pltpu.sync_copy(w_hbm.at[pl.ds(t0, TW)], w0_v)
        pltpu.sync_copy(w_hbm.at[pl.ds(N + t0, TW)], w1_v)

        nchunk = TW // TC
        sems = (sem0, sem1)

        def start(c):
            b = c % 2
            i0 = idx0_v[pl.ds(c * TC, TC)]
            i1 = idx1_v[pl.ds(c * TC, TC)]
            return (
                pltpu.async_copy(outs_hbm.at[i0], r0_v.at[b], sems[b]),
                pltpu.async_copy(outs_hbm.at[i1], r1_v.at[b], sems[b]),
            )

        cps = start(0)
        for c in range(nchunk):
            b = c % 2
            nxt = start(c + 1) if c + 1 < nchunk else None
            for cp in cps:
                cp.wait()
            w0c = w0_v[pl.ds(c * TC, TC)]
            w1c = w1_v[pl.ds(c * TC, TC)]

            def tok(i, carry):
                w0s = _splat_lane(w0c, i)
                w1s = _splat_lane(w1c, i)
                for j in range(D // 16):
                    sl = pl.ds(j * 16, 16)
                    y_v[i, sl] = w0s * r0_v[b, i, sl] + w1s * r1_v[b, i, sl]
                return carry

            lax.fori_loop(0, TC, tok, 0)
            pltpu.sync_copy(y_v, y_hbm.at[pl.ds(t0 + c * TC, TC)])
            cps = nxt

    return _combine


def kernel(hidden_states, gate_w, w_gate, w_up, w_down, sw_gate, sw_up, sw_down):
    b, s, d = hidden_states.shape
    x = hidden_states.reshape(b * s, d)

    w8, slot8, meta8 = _route(x, gate_w)
    slot_flat = slot8[:TOPK].reshape(NP)
    w_flat = w8[:TOPK].reshape(NP)
    be = meta8[0, :NBLK]
    act = meta8[1, :NBLK]

    slotc = jnp.transpose(slot8[:TOPK].reshape(TOPK, NW, 2, TW // 2),
                          (1, 2, 0, 3))
    xs = _sc_dispatch()(x, slotc)
    outs = _grouped_mlp(be, act, xs, w_gate, w_up, w_down)
    sh = _shared_mlp(x, sw_gate, sw_up, sw_down)
    yr = _sc_combine()(outs, slot_flat, w_flat)
    y = _final_add(sh, yr)
    return y.reshape(b, s, d)
